# baseline (speedup 1.0000x reference)
import functools

import jax
import jax.numpy as jnp
from jax import lax
from jax.experimental import pallas as pl
from jax.experimental.pallas import tpu as pltpu
from jax.experimental.pallas import tpu_sc as plsc

_N = 10000
_E = 320000
_D = 128
_H = 256
_G = 64

_NC = 2
_NS = 16


_NP = 10240


@functools.lru_cache(maxsize=None)
def _make_agg(Dh: int, e: int, esplit: bool = False):
  half = _NP // 2
  rpt = half // _NS
  ept = e // (2 * _NS) if esplit else e // _NS
  K = 128
  NB = 3
  n_groups = ept // (K * NB)
  tail = ept - n_groups * NB * K
  trem = tail % K
  RC = 80
  assert tail % 16 == 0 and rpt % RC == 0 and Dh == 128

  mesh = plsc.VectorSubcoreMesh(core_axis_name="c", subcore_axis_name="s")

  @functools.partial(
      pl.kernel,
      mesh=mesh,
      out_type=jax.ShapeDtypeStruct((2, _NP, Dh), jnp.float32),
      scratch_types=[
          pltpu.VMEM((ept,), jnp.int32),
          pltpu.VMEM((ept,), jnp.int32),
          [pltpu.VMEM((K,), jnp.int32) for _ in range(NB)],
          pltpu.VMEM((max(trem, 16),), jnp.int32),
          [pltpu.VMEM((K, Dh), jnp.float32) for _ in range(NB)],

          pltpu.VMEM_SHARED((half + 8, Dh), jnp.float32),
          [pltpu.SemaphoreType.DMA for _ in range(NB)],
          [pltpu.SemaphoreType.DMA for _ in range(NB)],

      ],
  )
  def agg_kernel(hC, srcx, dst, out, sstage, dstage, didxs, didx_tail,
                 rowbufs, acc, gsems, ssems):
    c = lax.axis_index("c")
    s = lax.axis_index("s")
    if esplit:
      eoff = c * (e // 2) + s * ept
      pltpu.sync_copy(srcx.at[pl.ds(eoff, ept)], sstage)
      pltpu.sync_copy(dst.at[pl.ds(eoff, ept)], dstage)
    else:
      pltpu.sync_copy(srcx.at[pl.ds(c * e + s * ept, ept)], sstage)
      pltpu.sync_copy(dst.at[pl.ds(s * ept, ept)], dstage)

    for p in range(2):
      r0 = s * rpt
      if esplit:
        @pl.when(c == 0)
        def _():
          pltpu.sync_copy(hC.at[pl.ds(p * half + r0, rpt)],
                          acc.at[pl.ds(r0, rpt)])

        @pl.when(c == 1)
        def _():
          for q in range(rpt // RC):
            pltpu.sync_copy(hC.at[pl.ds(_N, RC)],
                            acc.at[pl.ds(r0 + q * RC, RC)])
      else:
        pltpu.sync_copy(hC.at[pl.ds(c * _NP + p * half + r0, rpt)],
                        acc.at[pl.ds(r0, rpt)])
      plsc.subcore_barrier()

      def build_didx(didx, base, nk):
        for k in range(nk):
          d = dstage[pl.ds(base + k * 16, 16)]
          t = d - p * half
          valid = (t >= 0) & (t < half)
          didx[pl.ds(k * 16, 16)] = jnp.where(valid, t, half)

      def group(j, carry):
        base = j * (NB * K)
        cps = [
            pltpu.async_copy(hC.at[sstage.at[pl.ds(base + t * K, K)]],
                             rowbufs[t], gsems[t]) for t in range(NB)
        ]
        scs = []
        for t in range(NB):
          build_didx(didxs[t], base + t * K, K // 16)
          cps[t].wait()
          scs.append(pltpu.async_copy(rowbufs[t], acc.at[didxs[t]], ssems[t],
                                      add=True))
        for sc in scs:
          sc.wait()
        return carry

      lax.fori_loop(0, n_groups, group, 0)
      toff = n_groups * NB * K
      ti = 0
      rem = tail
      while rem > 0:
        sz = min(K, rem)
        didx = didxs[ti] if sz == K else didx_tail
        trow = rowbufs[ti] if sz == K else rowbufs[ti].at[pl.ds(0, sz)]
        cp = pltpu.async_copy(hC.at[sstage.at[pl.ds(toff, sz)]], trow,
                              gsems[ti])
        build_didx(didx, toff, sz // 16)
        cp.wait()
        pltpu.sync_copy(trow, acc.at[didx], add=True)
        toff += sz
        rem -= sz
        ti += 1
      plsc.subcore_barrier()
      r0 = s * rpt
      pltpu.sync_copy(acc.at[pl.ds(r0, rpt)],
                      out.at[c, pl.ds(p * half + r0, rpt)])
      plsc.subcore_barrier()

  return agg_kernel


@functools.lru_cache(maxsize=None)
def _make_dense(n: int, din: int, dh: int, mode: str):
  def body(h2pair, Wa, ba, Wb, bb, g, be, out):
    full = h2pair[...]
    if mode == 'split':
      h2 = jnp.concatenate([full[0, :n, :], full[1, :n, :]], axis=1)
    else:
      h2 = full[0, :n, :] + full[1, :n, :]
    a = jnp.dot(h2, Wa[...], preferred_element_type=jnp.float32)
    a = jnp.maximum(a + ba[...], 0.0)
    t = jnp.dot(a, Wb[...], preferred_element_type=jnp.float32) + bb[...]
    m = jnp.mean(t, axis=0, keepdims=True)
    d = t - m
    v = jnp.mean(d * d, axis=0, keepdims=True)
    out[...] = jnp.maximum(d * lax.rsqrt(v + 1e-5) * g[...] + be[...], 0.0)

  return pl.pallas_call(
      body, out_shape=jax.ShapeDtypeStruct((n, dh), jnp.float32))


@functools.lru_cache(maxsize=None)
def _make_pool_head(n: int, dh: int, g_: int, hmid: int, nout: int):
  def body(h, batch, Wc1, bc1, Wc2, bc2, out):
    ids = lax.broadcasted_iota(jnp.int32, (1, g_), 1)
    onehot = (batch[...] == ids).astype(jnp.float32)
    sums = lax.dot_general(onehot, h[...], (((0,), (0,)), ((), ())),
                           preferred_element_type=jnp.float32)
    counts = jnp.sum(onehot, axis=0, keepdims=True)
    pooled = sums / jnp.maximum(counts, 1.0).T
    z = jnp.dot(pooled, Wc1[...], preferred_element_type=jnp.float32)
    z = jnp.maximum(z + bc1[...], 0.0)
    out[...] = jnp.dot(z, Wc2[...],
                       preferred_element_type=jnp.float32) + bc2[...]

  return pl.pallas_call(
      body, out_shape=jax.ShapeDtypeStruct((g_, nout), jnp.float32))


def kernel(x, edge_index, batch, params):
  n, d = x.shape
  e = edge_index.shape[1]
  src = edge_index[0]
  dst = edge_index[1]
  srcx = jnp.concatenate([src, src + _NP])

  h = x
  for i in range(3):
    din = h.shape[1]
    split = din > 128
    Dh = din // 2 if split else din
    pad = jnp.zeros((_NP - n, Dh), jnp.float32)
    if split:
      hC = jnp.concatenate([h[:, :Dh], pad, h[:, Dh:], pad], axis=0)
    else:
      hC = jnp.concatenate([h, pad], axis=0)
    h2 = _make_agg(Dh, e, esplit=not split)(hC, srcx, dst)
    h = _make_dense(n, din, _H, 'split' if split else 'sum')(
        h2,
        params['W%da' % i], params['b%da' % i].reshape(1, -1),
        params['W%db' % i], params['b%db' % i].reshape(1, -1),
        params['g%d' % i].reshape(1, -1), params['be%d' % i].reshape(1, -1))

  return _make_pool_head(n, _H, _G, _H // 2, 2)(
      h, batch.reshape(n, 1).astype(jnp.int32),
      params['Wc1'], params['bc1'].reshape(1, -1),
      params['Wc2'], params['bc2'].reshape(1, -1))


# speedup vs baseline: 1.0246x; 1.0246x over previous
import functools

import jax
import jax.numpy as jnp
from jax import lax
from jax.experimental import pallas as pl
from jax.experimental.pallas import tpu as pltpu
from jax.experimental.pallas import tpu_sc as plsc

_N = 10000
_E = 320000
_D = 128
_H = 256
_G = 64

_NC = 2
_NS = 16


_NP = 10240


@functools.lru_cache(maxsize=None)
def _make_agg(Dh: int, e: int, esplit: bool = False):
  half = _NP // 2
  rpt = half // _NS
  ept = e // (2 * _NS) if esplit else e // _NS
  K = 128
  NB = 3
  n_groups = ept // (K * NB)
  tail = ept - n_groups * NB * K
  trem = tail % K
  RC = 80
  assert tail % 16 == 0 and rpt % RC == 0 and Dh == 128

  mesh = plsc.VectorSubcoreMesh(core_axis_name="c", subcore_axis_name="s")

  @functools.partial(
      pl.kernel,
      mesh=mesh,
      out_type=jax.ShapeDtypeStruct((2, _NP, Dh), jnp.float32),
      scratch_types=[
          pltpu.VMEM((ept,), jnp.int32),
          pltpu.VMEM((ept,), jnp.int32),
          [pltpu.VMEM((K,), jnp.int32) for _ in range(NB)],
          pltpu.VMEM((max(trem, 16),), jnp.int32),
          [pltpu.VMEM((K, Dh), jnp.float32) for _ in range(NB)],

          pltpu.VMEM_SHARED((half + 8, Dh), jnp.float32),
          [pltpu.SemaphoreType.DMA for _ in range(NB)],
          [pltpu.SemaphoreType.DMA for _ in range(NB)],

      ],
  )
  def agg_kernel(hC, srcx, dst, out, sstage, dstage, didxs, didx_tail,
                 rowbufs, acc, gsems, ssems):
    c = lax.axis_index("c")
    s = lax.axis_index("s")
    if esplit:
      eoff = c * (e // 2) + s * ept
      pltpu.sync_copy(srcx.at[pl.ds(eoff, ept)], sstage)
      pltpu.sync_copy(dst.at[pl.ds(eoff, ept)], dstage)
    else:
      pltpu.sync_copy(srcx.at[pl.ds(c * e + s * ept, ept)], sstage)
      pltpu.sync_copy(dst.at[pl.ds(s * ept, ept)], dstage)

    for p in range(2):
      r0 = s * rpt
      if esplit:
        @pl.when(c == 0)
        def _():
          pltpu.sync_copy(hC.at[pl.ds(p * half + r0, rpt)],
                          acc.at[pl.ds(r0, rpt)])

        @pl.when(c == 1)
        def _():
          for q in range(rpt // RC):
            pltpu.sync_copy(hC.at[pl.ds(_N, RC)],
                            acc.at[pl.ds(r0 + q * RC, RC)])
      else:
        pltpu.sync_copy(hC.at[pl.ds(c * _NP + p * half + r0, rpt)],
                        acc.at[pl.ds(r0, rpt)])
      plsc.subcore_barrier()

      def build_didx(didx, base, nk):
        for k in range(nk):
          d = dstage[pl.ds(base + k * 16, 16)]
          t = d - p * half
          valid = (t >= 0) & (t < half)
          didx[pl.ds(k * 16, 16)] = jnp.where(valid, t, half)

      def group(j, carry):
        base = j * (NB * K)
        cps = [
            pltpu.async_copy(hC.at[sstage.at[pl.ds(base + t * K, K)]],
                             rowbufs[t], gsems[t]) for t in range(NB)
        ]
        scs = []
        for t in range(NB):
          build_didx(didxs[t], base + t * K, K // 16)
          cps[t].wait()
          scs.append(pltpu.async_copy(rowbufs[t], acc.at[didxs[t]], ssems[t],
                                      add=True))
        for sc in scs:
          sc.wait()
        return carry

      lax.fori_loop(0, n_groups, group, 0)
      toff = n_groups * NB * K
      ti = 0
      rem = tail
      while rem > 0:
        sz = min(K, rem)
        didx = didxs[ti] if sz == K else didx_tail
        trow = rowbufs[ti] if sz == K else rowbufs[ti].at[pl.ds(0, sz)]
        cp = pltpu.async_copy(hC.at[sstage.at[pl.ds(toff, sz)]], trow,
                              gsems[ti])
        build_didx(didx, toff, sz // 16)
        cp.wait()
        pltpu.sync_copy(trow, acc.at[didx], add=True)
        toff += sz
        rem -= sz
        ti += 1
      plsc.subcore_barrier()
      r0 = s * rpt
      pltpu.sync_copy(acc.at[pl.ds(r0, rpt)],
                      out.at[c, pl.ds(p * half + r0, rpt)])
      plsc.subcore_barrier()

  return agg_kernel


@functools.lru_cache(maxsize=None)
def _make_dense(n: int, din: int, dh: int, mode: str, out_hc: bool):
  def body(h2pair, Wa, ba, Wb, bb, g, be, out):
    full = h2pair[...]
    if mode == 'split':
      h2 = jnp.concatenate([full[0, :n, :], full[1, :n, :]], axis=1)
    else:
      h2 = full[0, :n, :] + full[1, :n, :]
    a = jnp.dot(h2, Wa[...], preferred_element_type=jnp.float32)
    a = jnp.maximum(a + ba[...], 0.0)
    t = jnp.dot(a, Wb[...], preferred_element_type=jnp.float32) + bb[...]
    m = jnp.mean(t, axis=0, keepdims=True)
    d = t - m
    v = jnp.mean(d * d, axis=0, keepdims=True)
    r = jnp.maximum(d * lax.rsqrt(v + 1e-5) * g[...] + be[...], 0.0)
    if out_hc:
      z = jnp.zeros((_NP - n, dh // 2), jnp.float32)
      out[...] = jnp.concatenate(
          [r[:, :dh // 2], z, r[:, dh // 2:], z], axis=0)
    else:
      out[...] = r

  oshape = (2 * _NP, dh // 2) if out_hc else (n, dh)
  return pl.pallas_call(
      body, out_shape=jax.ShapeDtypeStruct(oshape, jnp.float32))


@functools.lru_cache(maxsize=None)
def _make_pool_head(n: int, dh: int, g_: int, hmid: int, nout: int):
  def body(h, batch, Wc1, bc1, Wc2, bc2, out):
    ids = lax.broadcasted_iota(jnp.int32, (1, g_), 1)
    onehot = (batch[...] == ids).astype(jnp.float32)
    sums = lax.dot_general(onehot, h[...], (((0,), (0,)), ((), ())),
                           preferred_element_type=jnp.float32)
    counts = jnp.sum(onehot, axis=0, keepdims=True)
    pooled = sums / jnp.maximum(counts, 1.0).T
    z = jnp.dot(pooled, Wc1[...], preferred_element_type=jnp.float32)
    z = jnp.maximum(z + bc1[...], 0.0)
    out[...] = jnp.dot(z, Wc2[...],
                       preferred_element_type=jnp.float32) + bc2[...]

  return pl.pallas_call(
      body, out_shape=jax.ShapeDtypeStruct((g_, nout), jnp.float32))


def kernel(x, edge_index, batch, params):
  n, d = x.shape
  e = edge_index.shape[1]
  src = edge_index[0]
  dst = edge_index[1]
  srcx = jnp.concatenate([src, src + _NP])

  hC = jnp.concatenate([x, jnp.zeros((_NP - n, d), jnp.float32)], axis=0)
  for i in range(3):
    din = d if i == 0 else _H
    split = din > 128
    Dh = din // 2 if split else din
    h2 = _make_agg(Dh, e, esplit=not split)(hC, srcx, dst)
    hC = _make_dense(n, din, _H, 'split' if split else 'sum', i < 2)(
        h2,
        params['W%da' % i], params['b%da' % i].reshape(1, -1),
        params['W%db' % i], params['b%db' % i].reshape(1, -1),
        params['g%d' % i].reshape(1, -1), params['be%d' % i].reshape(1, -1))

  return _make_pool_head(n, _H, _G, _H // 2, 2)(
      hC, batch.reshape(n, 1).astype(jnp.int32),
      params['Wc1'], params['bc1'].reshape(1, -1),
      params['Wc2'], params['bc2'].reshape(1, -1))


# speedup vs baseline: 1.0360x; 1.0112x over previous
import functools

import jax
import jax.numpy as jnp
from jax import lax
from jax.experimental import pallas as pl
from jax.experimental.pallas import tpu as pltpu
from jax.experimental.pallas import tpu_sc as plsc

_N = 10000
_E = 320000
_D = 128
_H = 256
_G = 64

_NC = 2
_NS = 16


_NP = 10240


@functools.lru_cache(maxsize=None)
def _make_agg(Dh: int, e: int, esplit: bool = False):
  half = _NP // 2
  rpt = half // _NS
  ept = e // (2 * _NS) if esplit else e // _NS
  K = 96
  NB = 4
  n_groups = ept // (K * NB)
  tail = ept - n_groups * NB * K
  trem = tail % K
  RC = 80
  assert tail % 16 == 0 and rpt % RC == 0 and Dh == 128

  mesh = plsc.VectorSubcoreMesh(core_axis_name="c", subcore_axis_name="s")

  @functools.partial(
      pl.kernel,
      mesh=mesh,
      out_type=jax.ShapeDtypeStruct((2, _NP, Dh), jnp.float32),
      scratch_types=[
          pltpu.VMEM((ept,), jnp.int32),
          pltpu.VMEM((ept,), jnp.int32),
          [pltpu.VMEM((K,), jnp.int32) for _ in range(NB)],
          pltpu.VMEM((max(trem, 16),), jnp.int32),
          [pltpu.VMEM((K, Dh), jnp.float32) for _ in range(NB)],

          pltpu.VMEM_SHARED((half + 8, Dh), jnp.float32),
          [pltpu.SemaphoreType.DMA for _ in range(NB)],
          [pltpu.SemaphoreType.DMA for _ in range(NB)],

      ],
  )
  def agg_kernel(hC, srcx, dst, out, sstage, dstage, didxs, didx_tail,
                 rowbufs, acc, gsems, ssems):
    c = lax.axis_index("c")
    s = lax.axis_index("s")
    if esplit:
      eoff = c * (e // 2) + s * ept
      pltpu.sync_copy(srcx.at[pl.ds(eoff, ept)], sstage)
      pltpu.sync_copy(dst.at[pl.ds(eoff, ept)], dstage)
    else:
      pltpu.sync_copy(srcx.at[pl.ds(c * e + s * ept, ept)], sstage)
      pltpu.sync_copy(dst.at[pl.ds(s * ept, ept)], dstage)

    for p in range(2):
      r0 = s * rpt
      if esplit:
        @pl.when(c == 0)
        def _():
          pltpu.sync_copy(hC.at[pl.ds(p * half + r0, rpt)],
                          acc.at[pl.ds(r0, rpt)])

        @pl.when(c == 1)
        def _():
          for q in range(rpt // RC):
            pltpu.sync_copy(hC.at[pl.ds(_N, RC)],
                            acc.at[pl.ds(r0 + q * RC, RC)])
      else:
        pltpu.sync_copy(hC.at[pl.ds(c * _NP + p * half + r0, rpt)],
                        acc.at[pl.ds(r0, rpt)])
      plsc.subcore_barrier()

      def build_didx(didx, base, nk):
        for k in range(nk):
          d = dstage[pl.ds(base + k * 16, 16)]
          t = d - p * half
          valid = (t >= 0) & (t < half)
          didx[pl.ds(k * 16, 16)] = jnp.where(valid, t, half)

      def group(j, carry):
        base = j * (NB * K)
        cps = [
            pltpu.async_copy(hC.at[sstage.at[pl.ds(base + t * K, K)]],
                             rowbufs[t], gsems[t]) for t in range(NB)
        ]
        scs = []
        for t in range(NB):
          build_didx(didxs[t], base + t * K, K // 16)
          cps[t].wait()
          scs.append(pltpu.async_copy(rowbufs[t], acc.at[didxs[t]], ssems[t],
                                      add=True))
        for sc in scs:
          sc.wait()
        return carry

      lax.fori_loop(0, n_groups, group, 0)
      toff = n_groups * NB * K
      ti = 0
      rem = tail
      while rem > 0:
        sz = min(K, rem)
        didx = didxs[ti] if sz == K else didx_tail
        trow = rowbufs[ti] if sz == K else rowbufs[ti].at[pl.ds(0, sz)]
        cp = pltpu.async_copy(hC.at[sstage.at[pl.ds(toff, sz)]], trow,
                              gsems[ti])
        build_didx(didx, toff, sz // 16)
        cp.wait()
        pltpu.sync_copy(trow, acc.at[didx], add=True)
        toff += sz
        rem -= sz
        ti += 1
      plsc.subcore_barrier()
      r0 = s * rpt
      pltpu.sync_copy(acc.at[pl.ds(r0, rpt)],
                      out.at[c, pl.ds(p * half + r0, rpt)])
      plsc.subcore_barrier()

  return agg_kernel


@functools.lru_cache(maxsize=None)
def _make_dense(n: int, din: int, dh: int, mode: str, out_hc: bool):
  def body(h2pair, Wa, ba, Wb, bb, g, be, out):
    full = h2pair[...]
    if mode == 'split':
      h2 = jnp.concatenate([full[0, :n, :], full[1, :n, :]], axis=1)
    else:
      h2 = full[0, :n, :] + full[1, :n, :]
    a = jnp.dot(h2, Wa[...], preferred_element_type=jnp.float32)
    a = jnp.maximum(a + ba[...], 0.0)
    t = jnp.dot(a, Wb[...], preferred_element_type=jnp.float32) + bb[...]
    m = jnp.mean(t, axis=0, keepdims=True)
    d = t - m
    v = jnp.mean(d * d, axis=0, keepdims=True)
    r = jnp.maximum(d * lax.rsqrt(v + 1e-5) * g[...] + be[...], 0.0)
    if out_hc:
      z = jnp.zeros((_NP - n, dh // 2), jnp.float32)
      out[...] = jnp.concatenate(
          [r[:, :dh // 2], z, r[:, dh // 2:], z], axis=0)
    else:
      out[...] = r

  oshape = (2 * _NP, dh // 2) if out_hc else (n, dh)
  return pl.pallas_call(
      body, out_shape=jax.ShapeDtypeStruct(oshape, jnp.float32))


@functools.lru_cache(maxsize=None)
def _make_pool_head(n: int, dh: int, g_: int, hmid: int, nout: int):
  def body(h, batch, Wc1, bc1, Wc2, bc2, out):
    ids = lax.broadcasted_iota(jnp.int32, (1, g_), 1)
    onehot = (batch[...] == ids).astype(jnp.float32)
    sums = lax.dot_general(onehot, h[...], (((0,), (0,)), ((), ())),
                           preferred_element_type=jnp.float32)
    counts = jnp.sum(onehot, axis=0, keepdims=True)
    pooled = sums / jnp.maximum(counts, 1.0).T
    z = jnp.dot(pooled, Wc1[...], preferred_element_type=jnp.float32)
    z = jnp.maximum(z + bc1[...], 0.0)
    out[...] = jnp.dot(z, Wc2[...],
                       preferred_element_type=jnp.float32) + bc2[...]

  return pl.pallas_call(
      body, out_shape=jax.ShapeDtypeStruct((g_, nout), jnp.float32))


def kernel(x, edge_index, batch, params):
  n, d = x.shape
  e = edge_index.shape[1]
  src = edge_index[0]
  dst = edge_index[1]
  srcx = jnp.concatenate([src, src + _NP])

  hC = jnp.concatenate([x, jnp.zeros((_NP - n, d), jnp.float32)], axis=0)
  for i in range(3):
    din = d if i == 0 else _H
    split = din > 128
    Dh = din // 2 if split else din
    h2 = _make_agg(Dh, e, esplit=not split)(hC, srcx, dst)
    hC = _make_dense(n, din, _H, 'split' if split else 'sum', i < 2)(
        h2,
        params['W%da' % i], params['b%da' % i].reshape(1, -1),
        params['W%db' % i], params['b%db' % i].reshape(1, -1),
        params['g%d' % i].reshape(1, -1), params['be%d' % i].reshape(1, -1))

  return _make_pool_head(n, _H, _G, _H // 2, 2)(
      hC, batch.reshape(n, 1).astype(jnp.int32),
      params['Wc1'], params['bc1'].reshape(1, -1),
      params['Wc2'], params['bc2'].reshape(1, -1))
